# baseline (speedup 1.0000x reference)
import numpy as np
import jax
import jax.numpy as jnp
from jax.experimental import pallas as pl
from jax.experimental.pallas import tpu as pltpu

_JNT = np.array([0, 5, 1, 9, 13, 17, 6, 2, 10, 14, 18, 7, 3, 11, 15, 19, 8, 4, 12, 16, 20])
_PERM = (_JNT[:, None] + np.arange(3)[None, :]).flatten()

_ROWS = 16384
_COLS = 63
_KSRC = 24
_BC = 16384
_GRID = _ROWS // _BC

_PSEL = np.zeros((_COLS, _KSRC), np.float32)
_PSEL[np.arange(_COLS), _PERM] = 1.0


def _body(p_ref, x_ref, o_ref, z_ref):
    o_ref[...] = jnp.dot(p_ref[...], x_ref[...], preferred_element_type=jnp.float32)
    z_ref[...] = jnp.zeros_like(z_ref)


def kernel(inputs):
    x_t = inputs.T
    x_t = pltpu.with_memory_space_constraint(x_t, pltpu.HBM)
    out_t, z_t = pl.pallas_call(
        _body,
        grid=(_GRID,),
        in_specs=[
            pl.BlockSpec((_COLS, _KSRC), lambda i: (0, 0)),
            pl.BlockSpec((_KSRC, _BC), lambda i: (0, i)),
        ],
        out_specs=[
            pl.BlockSpec((_COLS, _BC), lambda i: (0, i)),
            pl.BlockSpec((1, _BC), lambda i: (0, i)),
        ],
        out_shape=[
            jax.ShapeDtypeStruct((_COLS, _ROWS), jnp.float32),
            jax.ShapeDtypeStruct((1, _ROWS), jnp.float32),
        ],
        compiler_params=pltpu.CompilerParams(
            dimension_semantics=("parallel",),
        ),
    )(jnp.asarray(_PSEL), x_t)
    return (out_t.T, z_t.T)


# speedup vs baseline: 1.1083x; 1.1083x over previous
import numpy as np
import jax
import jax.numpy as jnp
from jax.experimental import pallas as pl
from jax.experimental.pallas import tpu as pltpu

_JNT = np.array([0, 5, 1, 9, 13, 17, 6, 2, 10, 14, 18, 7, 3, 11, 15, 19, 8, 4, 12, 16, 20])
_PERM = (_JNT[:, None] + np.arange(3)[None, :]).flatten()

_ROWS = 16384
_COLS = 63
_KSRC = 24
_BC = 8192
_GRID = _ROWS // _BC

_PSEL = np.zeros((_COLS, _KSRC), np.float32)
_PSEL[np.arange(_COLS), _PERM] = 1.0


def _body(p_ref, x_ref, o_ref, z_ref):
    o_ref[...] = jnp.dot(p_ref[...], x_ref[...], preferred_element_type=jnp.float32)
    z_ref[...] = jnp.zeros_like(z_ref)


def kernel(inputs):
    x_t = inputs.T
    x_t = pltpu.with_memory_space_constraint(x_t, pltpu.HBM)
    out_t, z_t = pl.pallas_call(
        _body,
        grid=(_GRID,),
        in_specs=[
            pl.BlockSpec((_COLS, _KSRC), lambda i: (0, 0)),
            pl.BlockSpec((_KSRC, _BC), lambda i: (0, i)),
        ],
        out_specs=[
            pl.BlockSpec((_COLS, _BC), lambda i: (0, i)),
            pl.BlockSpec((1, _BC), lambda i: (0, i)),
        ],
        out_shape=[
            jax.ShapeDtypeStruct((_COLS, _ROWS), jnp.float32),
            jax.ShapeDtypeStruct((1, _ROWS), jnp.float32),
        ],
        compiler_params=pltpu.CompilerParams(
            dimension_semantics=("parallel",),
        ),
    )(jnp.asarray(_PSEL), x_t)
    return (out_t.T, z_t.T)
